# baseline (speedup 1.0000x reference)
import math

import jax
import jax.numpy as jnp
from jax.experimental import pallas as pl

N_Q = 8
BINS = 1024
DIM = 256
CODE_DIM = 8
NCHAINS = 1


def _rvq_kernel(x_ref, pw_ref, pb_ref, pow_ref, pob_ref, cb_ref, c2h_ref,
                q_out_ref, codes_ref, loss_ref):
    Tt = x_ref.shape[2]
    H = Tt // NCHAINS
    row_iota = jax.lax.broadcasted_iota(jnp.int32, (BINS, H), 0)

    def stage(i, r):
        P = pw_ref[i]
        xp = jax.lax.dot_general(P, r, (((1,), (0,)), ((), ())),
                                 preferred_element_type=jnp.float32)
        xp = xp + pb_ref[i][:, None]
        s = jax.lax.dot_general(cb_ref[i], xp, (((1,), (0,)), ((), ())),
                                preferred_element_type=jnp.float32)
        sc = c2h_ref[i][:, None] - s
        idx = jnp.argmin(sc, axis=0)
        onehot = (row_iota == idx[None, :]).astype(jnp.float32)
        q = jax.lax.dot_general(cb_ref[i], onehot, (((0,), (0,)), ((), ())),
                                preferred_element_type=jnp.float32)
        lp = jnp.sum((q - xp) ** 2, axis=0)
        qo = jax.lax.dot_general(pow_ref[i], q, (((1,), (0,)), ((), ())),
                                 preferred_element_type=jnp.float32)
        qo = qo + pob_ref[i][:, None]
        return r - qo, idx, lp

    chains = []
    for h in range(NCHAINS):
        r = x_ref[0, :, h * H:(h + 1) * H]
        chains.append({"r": r, "idx": [], "lp": []})

    for i in range(N_Q):
        for st in chains:
            r, idx, lp = stage(i, st["r"])
            st["r"] = r
            st["idx"].append(idx)
            st["lp"].append(lp)

    b = pl.program_id(0)
    t = pl.program_id(1)
    for h, st in enumerate(chains):
        sl = pl.ds(h * H, H)
        q_out_ref[0, :, sl] = x_ref[0, :, sl] - st["r"]
        codes_ref[:, sl] = jnp.stack(st["idx"], axis=0)
        contrib = jnp.stack(st["lp"], axis=0)

        @pl.when(jnp.logical_and(b == 0, t == 0))
        def _init():
            loss_ref[:, sl] = contrib

        @pl.when(jnp.logical_or(b != 0, t != 0))
        def _acc():
            loss_ref[:, sl] = loss_ref[:, sl] + contrib


def kernel(x, frame_rate, proj_in_w, proj_in_b, proj_out_w, proj_out_b, codebooks):
    B, D, T = x.shape
    Tt = 2048
    grid = (B, T // Tt)

    c2h = 0.5 * jnp.sum(codebooks * codebooks, axis=-1)

    quantized, codes_tmp, loss_parts = pl.pallas_call(
        _rvq_kernel,
        grid=grid,
        in_specs=[
            pl.BlockSpec((1, D, Tt), lambda b, t: (b, 0, t)),
            pl.BlockSpec((N_Q, CODE_DIM, D), lambda b, t: (0, 0, 0)),
            pl.BlockSpec((N_Q, CODE_DIM), lambda b, t: (0, 0)),
            pl.BlockSpec((N_Q, D, CODE_DIM), lambda b, t: (0, 0, 0)),
            pl.BlockSpec((N_Q, D), lambda b, t: (0, 0)),
            pl.BlockSpec((N_Q, BINS, CODE_DIM), lambda b, t: (0, 0, 0)),
            pl.BlockSpec((N_Q, BINS), lambda b, t: (0, 0)),
        ],
        out_specs=[
            pl.BlockSpec((1, D, Tt), lambda b, t: (b, 0, t)),
            pl.BlockSpec((N_Q, Tt), lambda b, t: (0, b * (T // Tt) + t)),
            pl.BlockSpec((N_Q, Tt), lambda b, t: (0, 0)),
        ],
        out_shape=[
            jax.ShapeDtypeStruct((B, D, T), jnp.float32),
            jax.ShapeDtypeStruct((N_Q, B * T), jnp.int32),
            jax.ShapeDtypeStruct((N_Q, Tt), jnp.float32),
        ],
    )(x, proj_in_w, proj_in_b, proj_out_w, proj_out_b, codebooks, c2h)

    codes = codes_tmp.reshape(N_Q, B, T)
    commit_loss = jnp.sum(loss_parts, axis=1) / (B * T * CODE_DIM)
    bw = jnp.asarray(N_Q * math.log2(BINS) * frame_rate, x.dtype)
    return quantized, codes, bw, commit_loss


# speedup vs baseline: 1.3047x; 1.3047x over previous
import math

import jax
import jax.numpy as jnp
from jax.experimental import pallas as pl

N_Q = 8
BINS = 1024
DIM = 256
CODE_DIM = 8
NCHAINS = 1


def _rvq_kernel(x_ref, pw_ref, pb_ref, pow_ref, pob_ref, cb_ref, c2h_ref,
                cbt_ref, q_out_ref, codes_ref, loss_ref):
    Tt = x_ref.shape[2]
    H = Tt // NCHAINS

    def stage(i, r):
        P = pw_ref[i]
        xp = jax.lax.dot_general(P, r, (((1,), (0,)), ((), ())),
                                 preferred_element_type=jnp.float32)
        xp = xp + pb_ref[i][:, None]
        s = jax.lax.dot_general(cb_ref[i], xp, (((1,), (0,)), ((), ())),
                                preferred_element_type=jnp.float32)
        sc = c2h_ref[i][:, None] - s
        idx = jnp.argmin(sc, axis=0)
        lo = jnp.broadcast_to(jnp.bitwise_and(idx, 127)[None, :], (CODE_DIM, H))
        hi = jnp.broadcast_to(jnp.right_shift(idx, 7)[None, :], (CODE_DIM, H))
        q = jnp.take_along_axis(cbt_ref[i, 0], lo, axis=1)
        for g in range(1, BINS // 128):
            qg = jnp.take_along_axis(cbt_ref[i, g], lo, axis=1)
            q = jnp.where(hi == g, qg, q)
        lp = jnp.sum((q - xp) ** 2, axis=0)
        qo = jax.lax.dot_general(pow_ref[i], q, (((1,), (0,)), ((), ())),
                                 preferred_element_type=jnp.float32)
        qo = qo + pob_ref[i][:, None]
        return r - qo, idx, lp

    chains = []
    for h in range(NCHAINS):
        r = x_ref[0, :, h * H:(h + 1) * H]
        chains.append({"r": r, "idx": [], "lp": []})

    for i in range(N_Q):
        for st in chains:
            r, idx, lp = stage(i, st["r"])
            st["r"] = r
            st["idx"].append(idx)
            st["lp"].append(lp)

    b = pl.program_id(0)
    t = pl.program_id(1)
    for h, st in enumerate(chains):
        sl = pl.ds(h * H, H)
        q_out_ref[0, :, sl] = x_ref[0, :, sl] - st["r"]
        codes_ref[:, sl] = jnp.stack(st["idx"], axis=0)
        contrib = jnp.stack(st["lp"], axis=0)

        @pl.when(jnp.logical_and(b == 0, t == 0))
        def _init():
            loss_ref[:, sl] = contrib

        @pl.when(jnp.logical_or(b != 0, t != 0))
        def _acc():
            loss_ref[:, sl] = loss_ref[:, sl] + contrib


def kernel(x, frame_rate, proj_in_w, proj_in_b, proj_out_w, proj_out_b, codebooks):
    B, D, T = x.shape
    Tt = 2048
    grid = (B, T // Tt)

    c2h = 0.5 * jnp.sum(codebooks * codebooks, axis=-1)
    cbt = jnp.transpose(codebooks.reshape(N_Q, BINS // 128, 128, CODE_DIM),
                        (0, 1, 3, 2))

    quantized, codes_tmp, loss_parts = pl.pallas_call(
        _rvq_kernel,
        grid=grid,
        in_specs=[
            pl.BlockSpec((1, D, Tt), lambda b, t: (b, 0, t)),
            pl.BlockSpec((N_Q, CODE_DIM, D), lambda b, t: (0, 0, 0)),
            pl.BlockSpec((N_Q, CODE_DIM), lambda b, t: (0, 0)),
            pl.BlockSpec((N_Q, D, CODE_DIM), lambda b, t: (0, 0, 0)),
            pl.BlockSpec((N_Q, D), lambda b, t: (0, 0)),
            pl.BlockSpec((N_Q, BINS, CODE_DIM), lambda b, t: (0, 0, 0)),
            pl.BlockSpec((N_Q, BINS), lambda b, t: (0, 0)),
            pl.BlockSpec((N_Q, BINS // 128, CODE_DIM, 128),
                         lambda b, t: (0, 0, 0, 0)),
        ],
        out_specs=[
            pl.BlockSpec((1, D, Tt), lambda b, t: (b, 0, t)),
            pl.BlockSpec((N_Q, Tt), lambda b, t: (0, b * (T // Tt) + t)),
            pl.BlockSpec((N_Q, Tt), lambda b, t: (0, 0)),
        ],
        out_shape=[
            jax.ShapeDtypeStruct((B, D, T), jnp.float32),
            jax.ShapeDtypeStruct((N_Q, B * T), jnp.int32),
            jax.ShapeDtypeStruct((N_Q, Tt), jnp.float32),
        ],
    )(x, proj_in_w, proj_in_b, proj_out_w, proj_out_b, codebooks, c2h, cbt)

    codes = codes_tmp.reshape(N_Q, B, T)
    commit_loss = jnp.sum(loss_parts, axis=1) / (B * T * CODE_DIM)
    bw = jnp.asarray(N_Q * math.log2(BINS) * frame_rate, x.dtype)
    return quantized, codes, bw, commit_loss
